# Initial kernel scaffold; baseline (speedup 1.0000x reference)
import jax
import jax.numpy as jnp
from jax.experimental import pallas as pl


def kernel(x, W, expert_bias):
    raise NotImplementedError("write your pallas kernel here")


# speedup vs baseline: 1.6813x; 1.6813x over previous
import functools

import jax
import jax.numpy as jnp
from jax.experimental import pallas as pl
from jax.experimental.pallas import tpu as pltpu

L_HEADS = 16
K_SEL = 2


def _router_kernel(x_ref, w_ref, b_ref, heads_ref, probs_ref, loss_ref,
                   acc_ref, *, n_steps, inv_total):
    step = pl.program_id(0)
    L = L_HEADS

    logits = jnp.dot(x_ref[...], w_ref[...],
                     preferred_element_type=jnp.float32)
    bl = logits + b_ref[0, :][None, :]

    iota = jax.lax.broadcasted_iota(jnp.int32, bl.shape, 1)
    m1 = jnp.max(bl, axis=-1, keepdims=True)
    i1 = jnp.min(jnp.where(bl == m1, iota, L), axis=-1, keepdims=True)
    sel1 = iota == i1
    bl2 = jnp.where(sel1, -jnp.inf, bl)
    m2 = jnp.max(bl2, axis=-1, keepdims=True)
    i2 = jnp.min(jnp.where(bl2 == m2, iota, L), axis=-1, keepdims=True)
    sel2 = iota == i2

    l1 = jnp.sum(jnp.where(sel1, logits, 0.0), axis=-1, keepdims=True)
    l2 = jnp.sum(jnp.where(sel2, logits, 0.0), axis=-1, keepdims=True)
    r = jnp.exp(l2 - l1)
    inv = 1.0 / (1.0 + r)

    heads_ref[...] = jnp.concatenate([i1, i2], axis=-1)
    probs_ref[...] = jnp.concatenate([inv, r * inv], axis=-1)

    cnt = jnp.sum((sel1 | sel2).astype(jnp.float32), axis=0)

    @pl.when(step == 0)
    def _():
        acc_ref[...] = jnp.zeros_like(acc_ref)

    acc_ref[0, :] += cnt

    @pl.when(step == n_steps - 1)
    def _():
        freqs = acc_ref[...] * inv_total
        loss_ref[...] = jnp.sum((freqs - 1.0 / L) ** 2, axis=-1, keepdims=True)


def kernel(x, W, expert_bias):
    B, N, H = x.shape
    L = W.shape[1]
    K = K_SEL
    tokens = B * N
    T = 1024
    n_steps = tokens // T

    xf = x.reshape(tokens, H)
    bias2d = expert_bias.reshape(1, L)

    heads, probs, loss = pl.pallas_call(
        functools.partial(_router_kernel, n_steps=n_steps,
                          inv_total=1.0 / (tokens * K)),
        grid=(n_steps,),
        in_specs=[
            pl.BlockSpec((T, H), lambda i: (i, 0)),
            pl.BlockSpec((H, L), lambda i: (0, 0)),
            pl.BlockSpec((1, L), lambda i: (0, 0)),
        ],
        out_specs=[
            pl.BlockSpec((T, K), lambda i: (i, 0)),
            pl.BlockSpec((T, K), lambda i: (i, 0)),
            pl.BlockSpec((1, 1), lambda i: (0, 0)),
        ],
        out_shape=[
            jax.ShapeDtypeStruct((tokens, K), jnp.int32),
            jax.ShapeDtypeStruct((tokens, K), jnp.float32),
            jax.ShapeDtypeStruct((1, 1), jnp.float32),
        ],
        scratch_shapes=[pltpu.VMEM((1, L), jnp.float32)],
    )(xf, W, bias2d)

    return (heads.reshape(B, N, K), probs.reshape(B, N, K),
            loss.reshape(()))


# speedup vs baseline: 1.7143x; 1.0197x over previous
import functools

import jax
import jax.numpy as jnp
from jax.experimental import pallas as pl
from jax.experimental.pallas import tpu as pltpu

L_HEADS = 16
K_SEL = 2


def _router_kernel(x_ref, w_ref, b_ref, heads_ref, probs_ref, loss_ref,
                   acc_ref, *, n_steps, inv_total):
    step = pl.program_id(0)
    L = L_HEADS

    logits = jnp.dot(x_ref[...], w_ref[...],
                     preferred_element_type=jnp.float32)
    bl = logits + b_ref[0, :][None, :]

    iota = jax.lax.broadcasted_iota(jnp.int32, bl.shape, 1)
    m1 = jnp.max(bl, axis=-1, keepdims=True)
    i1 = jnp.min(jnp.where(bl == m1, iota, L), axis=-1, keepdims=True)
    sel1 = iota == i1
    bl2 = jnp.where(sel1, -jnp.inf, bl)
    m2 = jnp.max(bl2, axis=-1, keepdims=True)
    i2 = jnp.min(jnp.where(bl2 == m2, iota, L), axis=-1, keepdims=True)
    sel2 = iota == i2

    l1 = jnp.sum(jnp.where(sel1, logits, 0.0), axis=-1, keepdims=True)
    l2 = jnp.sum(jnp.where(sel2, logits, 0.0), axis=-1, keepdims=True)
    r = jnp.exp(l2 - l1)
    inv = 1.0 / (1.0 + r)

    heads_ref[...] = jnp.concatenate([i1, i2], axis=-1)
    probs_ref[...] = jnp.concatenate([inv, r * inv], axis=-1)

    cnt = jnp.sum((sel1 | sel2).astype(jnp.float32), axis=0)

    @pl.when(step == 0)
    def _():
        acc_ref[...] = jnp.zeros_like(acc_ref)

    acc_ref[0, :] += cnt

    @pl.when(step == n_steps - 1)
    def _():
        freqs = acc_ref[...] * inv_total
        loss_ref[...] = jnp.sum((freqs - 1.0 / L) ** 2, axis=-1, keepdims=True)


def kernel(x, W, expert_bias):
    B, N, H = x.shape
    L = W.shape[1]
    K = K_SEL
    tokens = B * N
    T = 2048
    n_steps = tokens // T

    xf = x.reshape(tokens, H)
    bias2d = expert_bias.reshape(1, L)

    heads, probs, loss = pl.pallas_call(
        functools.partial(_router_kernel, n_steps=n_steps,
                          inv_total=1.0 / (tokens * K)),
        grid=(n_steps,),
        in_specs=[
            pl.BlockSpec((T, H), lambda i: (i, 0)),
            pl.BlockSpec((H, L), lambda i: (0, 0)),
            pl.BlockSpec((1, L), lambda i: (0, 0)),
        ],
        out_specs=[
            pl.BlockSpec((T, K), lambda i: (i, 0)),
            pl.BlockSpec((T, K), lambda i: (i, 0)),
            pl.BlockSpec((1, 1), lambda i: (0, 0)),
        ],
        out_shape=[
            jax.ShapeDtypeStruct((tokens, K), jnp.int32),
            jax.ShapeDtypeStruct((tokens, K), jnp.float32),
            jax.ShapeDtypeStruct((1, 1), jnp.float32),
        ],
        scratch_shapes=[pltpu.VMEM((1, L), jnp.float32)],
    )(xf, W, bias2d)

    return (heads.reshape(B, N, K), probs.reshape(B, N, K),
            loss.reshape(()))


# speedup vs baseline: 2.5756x; 1.5024x over previous
import functools

import jax
import jax.numpy as jnp
from jax.experimental import pallas as pl
from jax.experimental.pallas import tpu as pltpu

L_HEADS = 16
K_SEL = 2


def _router_kernel(x_ref, w_ref, b_ref, heads_ref, probs_ref, loss_ref,
                   acc_ref, *, n_steps, inv_total):
    step = pl.program_id(0)
    L = L_HEADS

    lt = jax.lax.dot_general(w_ref[...], x_ref[...],
                             (((1,), (1,)), ((), ())),
                             preferred_element_type=jnp.float32)
    bl = lt + b_ref[...]

    iota = jax.lax.broadcasted_iota(jnp.int32, bl.shape, 0)
    m1 = jnp.max(bl, axis=0, keepdims=True)
    i1 = jnp.min(jnp.where(bl == m1, iota, L), axis=0, keepdims=True)
    sel1 = iota == i1
    bl2 = jnp.where(sel1, -jnp.inf, bl)
    m2 = jnp.max(bl2, axis=0, keepdims=True)
    i2 = jnp.min(jnp.where(bl2 == m2, iota, L), axis=0, keepdims=True)
    sel2 = iota == i2

    l1 = jnp.sum(jnp.where(sel1, lt, 0.0), axis=0, keepdims=True)
    l2 = jnp.sum(jnp.where(sel2, lt, 0.0), axis=0, keepdims=True)
    r = jnp.exp(l2 - l1)
    inv = 1.0 / (1.0 + r)

    heads_ref[...] = jnp.concatenate([i1, i2], axis=0)
    probs_ref[...] = jnp.concatenate([inv, r * inv], axis=0)

    cnt = jnp.sum((sel1 | sel2).astype(jnp.float32), axis=1,
                  keepdims=True)

    @pl.when(step == 0)
    def _():
        acc_ref[...] = jnp.zeros_like(acc_ref)

    acc_ref[...] += cnt

    @pl.when(step == n_steps - 1)
    def _():
        freqs = acc_ref[...] * inv_total
        loss_ref[...] = jnp.sum((freqs - 1.0 / L) ** 2, axis=0,
                                keepdims=True)


def kernel(x, W, expert_bias):
    B, N, H = x.shape
    L = W.shape[1]
    K = K_SEL
    tokens = B * N
    T = 1024
    n_steps = tokens // T

    xf = x.reshape(tokens, H)
    wt = W.T
    bias2d = expert_bias.reshape(L, 1)

    heads_t, probs_t, loss = pl.pallas_call(
        functools.partial(_router_kernel, n_steps=n_steps,
                          inv_total=1.0 / (tokens * K)),
        grid=(n_steps,),
        in_specs=[
            pl.BlockSpec((T, H), lambda i: (i, 0)),
            pl.BlockSpec((L, H), lambda i: (0, 0)),
            pl.BlockSpec((L, 1), lambda i: (0, 0)),
        ],
        out_specs=[
            pl.BlockSpec((K, T), lambda i: (0, i)),
            pl.BlockSpec((K, T), lambda i: (0, i)),
            pl.BlockSpec((1, 1), lambda i: (0, 0)),
        ],
        out_shape=[
            jax.ShapeDtypeStruct((K, tokens), jnp.int32),
            jax.ShapeDtypeStruct((K, tokens), jnp.float32),
            jax.ShapeDtypeStruct((1, 1), jnp.float32),
        ],
        scratch_shapes=[pltpu.VMEM((L, 1), jnp.float32)],
    )(xf, wt, bias2d)

    heads = heads_t.T.reshape(B, N, K)
    probs = probs_t.T.reshape(B, N, K)
    return (heads, probs, loss.reshape(()))
